# baseline (speedup 1.0000x reference)
import functools
import math

import jax
import jax.numpy as jnp
from jax import lax
from jax.experimental import pallas as pl
from jax.experimental.pallas import tpu as pltpu
from jax.experimental.pallas import tpu_sc as plsc

B, C, D = 1024, 100000, 128
S = 30.0
M = 0.3
COS_M = math.cos(M)
SIN_M = math.sin(M)
TH = math.cos(math.pi - M)
MM = math.sin(math.pi - M) * M

N_BLK = 5120
CHUNK = 1024
K_GRID = -(-C // N_BLK)

SC_NC, SC_NS = 2, 16
SC_NW = SC_NC * SC_NS
B_PER_W = B // SC_NW


def _sc_gather_body(table_hbm, idx_hbm, out_hbm, idx_v, rows_v, sem):
    wid = lax.axis_index("s") * SC_NC + lax.axis_index("c")
    base = wid * B_PER_W
    pltpu.sync_copy(idx_hbm.at[pl.ds(base, B_PER_W)], idx_v)
    pltpu.async_copy(table_hbm.at[idx_v], rows_v, sem).wait()
    pltpu.sync_copy(rows_v, out_hbm.at[pl.ds(base, B_PER_W)])


_sc_gather = functools.partial(
    pl.kernel,
    out_type=jax.ShapeDtypeStruct((B, D), jnp.float32),
    mesh=plsc.VectorSubcoreMesh(core_axis_name="c", subcore_axis_name="s"),
    scratch_types=[
        pltpu.VMEM((B_PER_W,), jnp.int32),
        pltpu.VMEM((B_PER_W, D), jnp.float32),
        pltpu.SemaphoreType.DMA,
    ],
)(_sc_gather_body)


LOG2E = math.log2(math.e)


def _oim_body(x_ref, feats_ref, ft_ref, out_ref, xs_scr, se_scr):
    i = pl.program_id(0)

    @pl.when(i == 0)
    def _init():
        x = x_ref[...]
        nrm = jnp.sqrt(jnp.sum(x * x, axis=1, keepdims=True))
        xs_scr[...] = (x * ((S * LOG2E) / jnp.clip(nrm, 1e-12))
                       ).astype(jnp.bfloat16)
        se_scr[...] = jnp.zeros_like(se_scr)

    def _tree_sum_128(v):
        parts = [v[:, k * 128:(k + 1) * 128] for k in range(v.shape[1] // 128)]
        while len(parts) > 1:
            nxt = [parts[k] + parts[k + 1] for k in range(0, len(parts) - 1, 2)]
            if len(parts) % 2:
                nxt.append(parts[-1])
            parts = nxt
        return parts[0]

    fb = feats_ref[...].astype(jnp.bfloat16)
    xs = xs_scr[...]
    es = []
    for c in range(N_BLK // CHUNK):
        yc = lax.dot_general(xs, fb[c * CHUNK:(c + 1) * CHUNK, :],
                             (((1,), (1,)), ((), ())),
                             preferred_element_type=jnp.float32)
        es.append(jnp.exp2(yc.astype(jnp.bfloat16)))

    @pl.when(i < K_GRID - 1)
    def _full_block():
        acc = _tree_sum_128(es[0])
        for ec in es[1:]:
            acc = acc + _tree_sum_128(ec)
        se_scr[...] += acc.astype(jnp.float32)

    @pl.when(i == K_GRID - 1)
    def _ragged_block():
        acc = None
        for c, ec in enumerate(es):
            col = (i * N_BLK + c * CHUNK
                   + lax.broadcasted_iota(jnp.int32, (B, CHUNK), 1))
            part = _tree_sum_128(jnp.where(col < C, ec,
                                           jnp.bfloat16(0.0)))
            acc = part if acc is None else acc + part
        se_scr[...] += acc.astype(jnp.float32)

    @pl.when(i == K_GRID - 1)
    def _epilogue():
        x = x_ref[...]
        xn = x / jnp.clip(jnp.sqrt(jnp.sum(x * x, axis=1, keepdims=True)),
                          1e-12)
        cos_t = jnp.sum(xn * ft_ref[...], axis=1, keepdims=True)
        sine = jnp.sqrt(jnp.clip(1.0 - cos_t * cos_t, 0.0, 1.0))
        phi = cos_t * COS_M - sine * SIN_M
        phi = jnp.where(cos_t > TH, phi, cos_t - MM)
        se = (jnp.sum(se_scr[...], axis=1, keepdims=True)
              + jnp.exp(S * phi) - jnp.exp(S * cos_t))
        lse = jnp.log(se)
        loss = jnp.mean(lse - S * phi)
        out_ref[...] = jnp.full((1, 1), loss, dtype=jnp.float32)


def kernel(inputs, targets, features):
    ft = _sc_gather(features, targets)
    out = pl.pallas_call(
        _oim_body,
        grid=(K_GRID,),
        in_specs=[
            pl.BlockSpec((B, D), lambda i: (0, 0)),
            pl.BlockSpec((N_BLK, D), lambda i: (i, 0)),
            pl.BlockSpec((B, D), lambda i: (0, 0)),
        ],
        out_specs=pl.BlockSpec((1, 1), lambda i: (0, 0)),
        out_shape=jax.ShapeDtypeStruct((1, 1), jnp.float32),
        scratch_shapes=[
            pltpu.VMEM((B, D), jnp.bfloat16),
            pltpu.VMEM((B, 128), jnp.float32),
        ],
        compiler_params=pltpu.CompilerParams(
            dimension_semantics=("arbitrary",),
        ),
    )(inputs, features, ft)
    return out[0, 0]


# speedup vs baseline: 1.0007x; 1.0007x over previous
import functools
import math

import jax
import jax.numpy as jnp
from jax import lax
from jax.experimental import pallas as pl
from jax.experimental.pallas import tpu as pltpu
from jax.experimental.pallas import tpu_sc as plsc

B, C, D = 1024, 100000, 128
S = 30.0
M = 0.3
COS_M = math.cos(M)
SIN_M = math.sin(M)
TH = math.cos(math.pi - M)
MM = math.sin(math.pi - M) * M

N_BLK = 5120
CHUNK = 512
K_GRID = -(-C // N_BLK)

SC_NC, SC_NS = 2, 16
SC_NW = SC_NC * SC_NS
B_PER_W = B // SC_NW


def _sc_gather_body(table_hbm, idx_hbm, out_hbm, idx_v, rows_v, sem):
    wid = lax.axis_index("s") * SC_NC + lax.axis_index("c")
    base = wid * B_PER_W
    pltpu.sync_copy(idx_hbm.at[pl.ds(base, B_PER_W)], idx_v)
    pltpu.async_copy(table_hbm.at[idx_v], rows_v, sem).wait()
    pltpu.sync_copy(rows_v, out_hbm.at[pl.ds(base, B_PER_W)])


_sc_gather = functools.partial(
    pl.kernel,
    out_type=jax.ShapeDtypeStruct((B, D), jnp.float32),
    mesh=plsc.VectorSubcoreMesh(core_axis_name="c", subcore_axis_name="s"),
    scratch_types=[
        pltpu.VMEM((B_PER_W,), jnp.int32),
        pltpu.VMEM((B_PER_W, D), jnp.float32),
        pltpu.SemaphoreType.DMA,
    ],
)(_sc_gather_body)


LOG2E = math.log2(math.e)


def _oim_body(x_ref, feats_ref, ft_ref, out_ref, xs_scr, se_scr):
    i = pl.program_id(0)

    @pl.when(i == 0)
    def _init():
        x = x_ref[...]
        nrm = jnp.sqrt(jnp.sum(x * x, axis=1, keepdims=True))
        xs_scr[...] = (x * ((S * LOG2E) / jnp.clip(nrm, 1e-12))
                       ).astype(jnp.bfloat16)
        se_scr[...] = jnp.zeros_like(se_scr)

    def _tree_sum_128(v):
        parts = [v[:, k * 128:(k + 1) * 128] for k in range(v.shape[1] // 128)]
        while len(parts) > 1:
            nxt = [parts[k] + parts[k + 1] for k in range(0, len(parts) - 1, 2)]
            if len(parts) % 2:
                nxt.append(parts[-1])
            parts = nxt
        return parts[0]

    fb = feats_ref[...].astype(jnp.bfloat16)
    xs = xs_scr[...]
    es = []
    for c in range(N_BLK // CHUNK):
        yc = lax.dot_general(xs, fb[c * CHUNK:(c + 1) * CHUNK, :],
                             (((1,), (1,)), ((), ())),
                             preferred_element_type=jnp.float32)
        es.append(jnp.exp2(yc.astype(jnp.bfloat16)))

    @pl.when(i < K_GRID - 1)
    def _full_block():
        acc = _tree_sum_128(es[0])
        for ec in es[1:]:
            acc = acc + _tree_sum_128(ec)
        se_scr[...] += acc.astype(jnp.float32)

    @pl.when(i == K_GRID - 1)
    def _ragged_block():
        acc = None
        for c, ec in enumerate(es):
            col = (i * N_BLK + c * CHUNK
                   + lax.broadcasted_iota(jnp.int32, (B, CHUNK), 1))
            part = _tree_sum_128(jnp.where(col < C, ec,
                                           jnp.bfloat16(0.0)))
            acc = part if acc is None else acc + part
        se_scr[...] += acc.astype(jnp.float32)

    @pl.when(i == K_GRID - 1)
    def _epilogue():
        x = x_ref[...]
        xn = x / jnp.clip(jnp.sqrt(jnp.sum(x * x, axis=1, keepdims=True)),
                          1e-12)
        cos_t = jnp.sum(xn * ft_ref[...], axis=1, keepdims=True)
        sine = jnp.sqrt(jnp.clip(1.0 - cos_t * cos_t, 0.0, 1.0))
        phi = cos_t * COS_M - sine * SIN_M
        phi = jnp.where(cos_t > TH, phi, cos_t - MM)
        se = (jnp.sum(se_scr[...], axis=1, keepdims=True)
              + jnp.exp(S * phi) - jnp.exp(S * cos_t))
        lse = jnp.log(se)
        loss = jnp.mean(lse - S * phi)
        out_ref[...] = jnp.full((1, 1), loss, dtype=jnp.float32)


def kernel(inputs, targets, features):
    ft = _sc_gather(features, targets)
    out = pl.pallas_call(
        _oim_body,
        grid=(K_GRID,),
        in_specs=[
            pl.BlockSpec((B, D), lambda i: (0, 0)),
            pl.BlockSpec((N_BLK, D), lambda i: (i, 0)),
            pl.BlockSpec((B, D), lambda i: (0, 0)),
        ],
        out_specs=pl.BlockSpec((1, 1), lambda i: (0, 0)),
        out_shape=jax.ShapeDtypeStruct((1, 1), jnp.float32),
        scratch_shapes=[
            pltpu.VMEM((B, D), jnp.bfloat16),
            pltpu.VMEM((B, 128), jnp.float32),
        ],
        compiler_params=pltpu.CompilerParams(
            dimension_semantics=("arbitrary",),
        ),
    )(inputs, features, ft)
    return out[0, 0]


# speedup vs baseline: 1.0309x; 1.0302x over previous
import functools
import math

import jax
import jax.numpy as jnp
from jax import lax
from jax.experimental import pallas as pl
from jax.experimental.pallas import tpu as pltpu
from jax.experimental.pallas import tpu_sc as plsc

B, C, D = 1024, 100000, 128
S = 30.0
M = 0.3
COS_M = math.cos(M)
SIN_M = math.sin(M)
TH = math.cos(math.pi - M)
MM = math.sin(math.pi - M) * M

N_BLK = 5120
CHUNK = 512
K_GRID = -(-C // N_BLK)

SC_NC, SC_NS = 2, 16
SC_NW = SC_NC * SC_NS
B_PER_W = B // SC_NW


def _sc_gather_body(table_hbm, idx_hbm, out_hbm, idx_v, rows_v, sem):
    wid = lax.axis_index("s") * SC_NC + lax.axis_index("c")
    base = wid * B_PER_W
    pltpu.sync_copy(idx_hbm.at[pl.ds(base, B_PER_W)], idx_v)
    pltpu.async_copy(table_hbm.at[idx_v], rows_v, sem).wait()
    pltpu.sync_copy(rows_v, out_hbm.at[pl.ds(base, B_PER_W)])


_sc_gather = functools.partial(
    pl.kernel,
    out_type=jax.ShapeDtypeStruct((B, D), jnp.float32),
    mesh=plsc.VectorSubcoreMesh(core_axis_name="c", subcore_axis_name="s"),
    scratch_types=[
        pltpu.VMEM((B_PER_W,), jnp.int32),
        pltpu.VMEM((B_PER_W, D), jnp.float32),
        pltpu.SemaphoreType.DMA,
    ],
)(_sc_gather_body)


LOG2E = math.log2(math.e)


def _oim_body(x_ref, feats_ref, se_ref, xs_scr, se_scr):
    i = pl.program_id(0)

    @pl.when(i == 0)
    def _init():
        x = x_ref[...]
        nrm = jnp.sqrt(jnp.sum(x * x, axis=1, keepdims=True))
        xs_scr[...] = (x * ((S * LOG2E) / jnp.clip(nrm, 1e-12))
                       ).astype(jnp.bfloat16)
        se_scr[...] = jnp.zeros_like(se_scr)

    def _tree_sum_128(v):
        parts = [v[:, k * 128:(k + 1) * 128] for k in range(v.shape[1] // 128)]
        while len(parts) > 1:
            nxt = [parts[k] + parts[k + 1] for k in range(0, len(parts) - 1, 2)]
            if len(parts) % 2:
                nxt.append(parts[-1])
            parts = nxt
        return parts[0]

    fb = feats_ref[...].astype(jnp.bfloat16)
    xs = xs_scr[...]
    es = []
    for c in range(N_BLK // CHUNK):
        yc = lax.dot_general(xs, fb[c * CHUNK:(c + 1) * CHUNK, :],
                             (((1,), (1,)), ((), ())),
                             preferred_element_type=jnp.float32)
        es.append(jnp.exp2(yc.astype(jnp.bfloat16)))

    @pl.when(i < K_GRID - 1)
    def _full_block():
        acc = _tree_sum_128(es[0])
        for ec in es[1:]:
            acc = acc + _tree_sum_128(ec)
        se_scr[...] += acc.astype(jnp.float32)

    @pl.when(i == K_GRID - 1)
    def _ragged_block():
        acc = None
        for c, ec in enumerate(es):
            col = (i * N_BLK + c * CHUNK
                   + lax.broadcasted_iota(jnp.int32, (B, CHUNK), 1))
            part = _tree_sum_128(jnp.where(col < C, ec,
                                           jnp.bfloat16(0.0)))
            acc = part if acc is None else acc + part
        se_scr[...] += acc.astype(jnp.float32)

    @pl.when(i == K_GRID - 1)
    def _writeback():
        se_ref[...] = se_scr[...]


def _epilogue_body(x_ref, ft_ref, se_ref, out_ref):
    x = x_ref[...]
    xn = x / jnp.clip(jnp.sqrt(jnp.sum(x * x, axis=1, keepdims=True)),
                      1e-12)
    cos_t = jnp.sum(xn * ft_ref[...], axis=1, keepdims=True)
    sine = jnp.sqrt(jnp.clip(1.0 - cos_t * cos_t, 0.0, 1.0))
    phi = cos_t * COS_M - sine * SIN_M
    phi = jnp.where(cos_t > TH, phi, cos_t - MM)
    se = (jnp.sum(se_ref[...], axis=1, keepdims=True)
          + jnp.exp(S * phi) - jnp.exp(S * cos_t))
    lse = jnp.log(se)
    loss = jnp.mean(lse - S * phi)
    out_ref[...] = jnp.full((1, 1), loss, dtype=jnp.float32)


def kernel(inputs, targets, features):
    ft = _sc_gather(features, targets)
    se = pl.pallas_call(
        _oim_body,
        grid=(K_GRID,),
        in_specs=[
            pl.BlockSpec((B, D), lambda i: (0, 0)),
            pl.BlockSpec((N_BLK, D), lambda i: (i, 0)),
        ],
        out_specs=pl.BlockSpec((B, 128), lambda i: (0, 0)),
        out_shape=jax.ShapeDtypeStruct((B, 128), jnp.float32),
        scratch_shapes=[
            pltpu.VMEM((B, D), jnp.bfloat16),
            pltpu.VMEM((B, 128), jnp.float32),
        ],
        compiler_params=pltpu.CompilerParams(
            dimension_semantics=("arbitrary",),
        ),
    )(inputs, features)
    out = pl.pallas_call(
        _epilogue_body,
        out_shape=jax.ShapeDtypeStruct((1, 1), jnp.float32),
    )(inputs, ft, se)
    return out[0, 0]


# speedup vs baseline: 1.0473x; 1.0159x over previous
import functools
import math

import jax
import jax.numpy as jnp
from jax import lax
from jax.experimental import pallas as pl
from jax.experimental.pallas import tpu as pltpu
from jax.experimental.pallas import tpu_sc as plsc

B, C, D = 1024, 100000, 128
S = 30.0
M = 0.3
COS_M = math.cos(M)
SIN_M = math.sin(M)
TH = math.cos(math.pi - M)
MM = math.sin(math.pi - M) * M

N_BLK = 10240
CHUNK = 512
K_GRID = -(-C // N_BLK)

SC_NC, SC_NS = 2, 16
SC_NW = SC_NC * SC_NS
B_PER_W = B // SC_NW


def _sc_gather_body(table_hbm, idx_hbm, out_hbm, idx_v, rows_v, sem):
    wid = lax.axis_index("s") * SC_NC + lax.axis_index("c")
    base = wid * B_PER_W
    pltpu.sync_copy(idx_hbm.at[pl.ds(base, B_PER_W)], idx_v)
    pltpu.async_copy(table_hbm.at[idx_v], rows_v, sem).wait()
    pltpu.sync_copy(rows_v, out_hbm.at[pl.ds(base, B_PER_W)])


_sc_gather = functools.partial(
    pl.kernel,
    out_type=jax.ShapeDtypeStruct((B, D), jnp.float32),
    mesh=plsc.VectorSubcoreMesh(core_axis_name="c", subcore_axis_name="s"),
    scratch_types=[
        pltpu.VMEM((B_PER_W,), jnp.int32),
        pltpu.VMEM((B_PER_W, D), jnp.float32),
        pltpu.SemaphoreType.DMA,
    ],
)(_sc_gather_body)


LOG2E = math.log2(math.e)


def _oim_body(x_ref, feats_ref, se_ref, xs_scr, se_scr):
    i = pl.program_id(0)

    @pl.when(i == 0)
    def _init():
        x = x_ref[...]
        nrm = jnp.sqrt(jnp.sum(x * x, axis=1, keepdims=True))
        xs_scr[...] = (x * ((S * LOG2E) / jnp.clip(nrm, 1e-12))
                       ).astype(jnp.bfloat16)
        se_scr[...] = jnp.zeros_like(se_scr)

    def _tree_sum_128(v):
        parts = [v[:, k * 128:(k + 1) * 128] for k in range(v.shape[1] // 128)]
        while len(parts) > 1:
            nxt = [parts[k] + parts[k + 1] for k in range(0, len(parts) - 1, 2)]
            if len(parts) % 2:
                nxt.append(parts[-1])
            parts = nxt
        return parts[0]

    fb = feats_ref[...].astype(jnp.bfloat16)
    xs = xs_scr[...]
    es = []
    for c in range(N_BLK // CHUNK):
        yc = lax.dot_general(xs, fb[c * CHUNK:(c + 1) * CHUNK, :],
                             (((1,), (1,)), ((), ())),
                             preferred_element_type=jnp.float32)
        es.append(jnp.exp2(yc.astype(jnp.bfloat16)))

    @pl.when(i < K_GRID - 1)
    def _full_block():
        acc = _tree_sum_128(es[0])
        for ec in es[1:]:
            acc = acc + _tree_sum_128(ec)
        se_scr[...] += acc.astype(jnp.float32)

    @pl.when(i == K_GRID - 1)
    def _ragged_block():
        acc = None
        for c, ec in enumerate(es):
            col = (i * N_BLK + c * CHUNK
                   + lax.broadcasted_iota(jnp.int32, (B, CHUNK), 1))
            part = _tree_sum_128(jnp.where(col < C, ec,
                                           jnp.bfloat16(0.0)))
            acc = part if acc is None else acc + part
        se_scr[...] += acc.astype(jnp.float32)

    @pl.when(i == K_GRID - 1)
    def _writeback():
        se_ref[...] = se_scr[...]


def _epilogue_body(x_ref, ft_ref, se_ref, out_ref):
    x = x_ref[...]
    xn = x / jnp.clip(jnp.sqrt(jnp.sum(x * x, axis=1, keepdims=True)),
                      1e-12)
    cos_t = jnp.sum(xn * ft_ref[...], axis=1, keepdims=True)
    sine = jnp.sqrt(jnp.clip(1.0 - cos_t * cos_t, 0.0, 1.0))
    phi = cos_t * COS_M - sine * SIN_M
    phi = jnp.where(cos_t > TH, phi, cos_t - MM)
    se = (jnp.sum(se_ref[...], axis=1, keepdims=True)
          + jnp.exp(S * phi) - jnp.exp(S * cos_t))
    lse = jnp.log(se)
    loss = jnp.mean(lse - S * phi)
    out_ref[...] = jnp.full((1, 1), loss, dtype=jnp.float32)


def kernel(inputs, targets, features):
    ft = _sc_gather(features, targets)
    se = pl.pallas_call(
        _oim_body,
        grid=(K_GRID,),
        in_specs=[
            pl.BlockSpec((B, D), lambda i: (0, 0)),
            pl.BlockSpec((N_BLK, D), lambda i: (i, 0)),
        ],
        out_specs=pl.BlockSpec((B, 128), lambda i: (0, 0)),
        out_shape=jax.ShapeDtypeStruct((B, 128), jnp.float32),
        scratch_shapes=[
            pltpu.VMEM((B, D), jnp.bfloat16),
            pltpu.VMEM((B, 128), jnp.float32),
        ],
        compiler_params=pltpu.CompilerParams(
            dimension_semantics=("arbitrary",),
        ),
    )(inputs, features)
    out = pl.pallas_call(
        _epilogue_body,
        out_shape=jax.ShapeDtypeStruct((1, 1), jnp.float32),
    )(inputs, ft, se)
    return out[0, 0]


# speedup vs baseline: 1.1691x; 1.1162x over previous
import functools
import math

import jax
import jax.numpy as jnp
from jax import lax
from jax.experimental import pallas as pl
from jax.experimental.pallas import tpu as pltpu
from jax.experimental.pallas import tpu_sc as plsc

B, C, D = 1024, 100000, 128
S = 30.0
M = 0.3
COS_M = math.cos(M)
SIN_M = math.sin(M)
TH = math.cos(math.pi - M)
MM = math.sin(math.pi - M) * M

N_BLK = 10240
CHUNK = 512
K_GRID = -(-C // N_BLK)

SC_NC, SC_NS = 2, 16
SC_NW = SC_NC * SC_NS
B_PER_W = B // SC_NW


def _sc_gather_body(table_hbm, idx_hbm, out_hbm, idx_v, rows_v, sem):
    wid = lax.axis_index("s") * SC_NC + lax.axis_index("c")
    base = wid * B_PER_W
    pltpu.sync_copy(idx_hbm.at[pl.ds(base, B_PER_W)], idx_v)
    pltpu.async_copy(table_hbm.at[idx_v], rows_v, sem).wait()
    pltpu.sync_copy(rows_v, out_hbm.at[pl.ds(base, B_PER_W)])


_sc_gather = functools.partial(
    pl.kernel,
    out_type=jax.ShapeDtypeStruct((B, D), jnp.float32),
    mesh=plsc.VectorSubcoreMesh(core_axis_name="c", subcore_axis_name="s"),
    scratch_types=[
        pltpu.VMEM((B_PER_W,), jnp.int32),
        pltpu.VMEM((B_PER_W, D), jnp.float32),
        pltpu.SemaphoreType.DMA,
    ],
)(_sc_gather_body)


LOG2E = math.log2(math.e)


def _oim_body(x_ref, feats_ref, se_ref, xs_scr, se_scr):
    i = pl.program_id(0)

    @pl.when(i == 0)
    def _init():
        x = x_ref[...]
        nrm = jnp.sqrt(jnp.sum(x * x, axis=1, keepdims=True))
        xs_scr[...] = (x * ((S * LOG2E) / jnp.clip(nrm, 1e-12))
                       ).astype(jnp.bfloat16)
        se_scr[...] = jnp.zeros_like(se_scr)

    def _tree_sum_128(v):
        parts = [v[:, k * 128:(k + 1) * 128] for k in range(v.shape[1] // 128)]
        while len(parts) > 1:
            nxt = [parts[k] + parts[k + 1] for k in range(0, len(parts) - 1, 2)]
            if len(parts) % 2:
                nxt.append(parts[-1])
            parts = nxt
        return parts[0]

    fb = feats_ref[...].astype(jnp.bfloat16)
    xs = xs_scr[...]
    acc = None
    for c in range(N_BLK // CHUNK):
        yc = lax.dot_general(xs, fb[c * CHUNK:(c + 1) * CHUNK, :],
                             (((1,), (1,)), ((), ())),
                             preferred_element_type=jnp.float32)
        ec = jnp.exp2(yc.astype(jnp.bfloat16))
        if (K_GRID - 1) * N_BLK + (c + 1) * CHUNK > C:
            col = (i * N_BLK + c * CHUNK
                   + lax.broadcasted_iota(jnp.int32, (B, CHUNK), 1))
            ec = jnp.where(col < C, ec, jnp.bfloat16(0.0))
        part = _tree_sum_128(ec)
        acc = part if acc is None else acc + part
    se_scr[...] += acc.astype(jnp.float32)

    @pl.when(i == K_GRID - 1)
    def _writeback():
        se_ref[...] = se_scr[...]


def _epilogue_body(x_ref, ft_ref, se_ref, out_ref):
    x = x_ref[...]
    xn = x / jnp.clip(jnp.sqrt(jnp.sum(x * x, axis=1, keepdims=True)),
                      1e-12)
    cos_t = jnp.sum(xn * ft_ref[...], axis=1, keepdims=True)
    sine = jnp.sqrt(jnp.clip(1.0 - cos_t * cos_t, 0.0, 1.0))
    phi = cos_t * COS_M - sine * SIN_M
    phi = jnp.where(cos_t > TH, phi, cos_t - MM)
    se = (jnp.sum(se_ref[...], axis=1, keepdims=True)
          + jnp.exp(S * phi) - jnp.exp(S * cos_t))
    lse = jnp.log(se)
    loss = jnp.mean(lse - S * phi)
    out_ref[...] = jnp.full((1, 1), loss, dtype=jnp.float32)


def kernel(inputs, targets, features):
    ft = _sc_gather(features, targets)
    se = pl.pallas_call(
        _oim_body,
        grid=(K_GRID,),
        in_specs=[
            pl.BlockSpec((B, D), lambda i: (0, 0)),
            pl.BlockSpec((N_BLK, D), lambda i: (i, 0)),
        ],
        out_specs=pl.BlockSpec((B, 128), lambda i: (0, 0)),
        out_shape=jax.ShapeDtypeStruct((B, 128), jnp.float32),
        scratch_shapes=[
            pltpu.VMEM((B, D), jnp.bfloat16),
            pltpu.VMEM((B, 128), jnp.float32),
        ],
        compiler_params=pltpu.CompilerParams(
            dimension_semantics=("arbitrary",),
        ),
    )(inputs, features)
    out = pl.pallas_call(
        _epilogue_body,
        out_shape=jax.ShapeDtypeStruct((1, 1), jnp.float32),
    )(inputs, ft, se)
    return out[0, 0]


# speedup vs baseline: 1.1709x; 1.0015x over previous
import functools
import math

import jax
import jax.numpy as jnp
from jax import lax
from jax.experimental import pallas as pl
from jax.experimental.pallas import tpu as pltpu
from jax.experimental.pallas import tpu_sc as plsc

B, C, D = 1024, 100000, 128
S = 30.0
M = 0.3
COS_M = math.cos(M)
SIN_M = math.sin(M)
TH = math.cos(math.pi - M)
MM = math.sin(math.pi - M) * M

N_BLK = 20480
CHUNK = 512
K_GRID = -(-C // N_BLK)

SC_NC, SC_NS = 2, 16
SC_NW = SC_NC * SC_NS
B_PER_W = B // SC_NW


def _sc_gather_body(table_hbm, idx_hbm, out_hbm, idx_v, rows_v, sem):
    wid = lax.axis_index("s") * SC_NC + lax.axis_index("c")
    base = wid * B_PER_W
    pltpu.sync_copy(idx_hbm.at[pl.ds(base, B_PER_W)], idx_v)
    pltpu.async_copy(table_hbm.at[idx_v], rows_v, sem).wait()
    pltpu.sync_copy(rows_v, out_hbm.at[pl.ds(base, B_PER_W)])


_sc_gather = functools.partial(
    pl.kernel,
    out_type=jax.ShapeDtypeStruct((B, D), jnp.float32),
    mesh=plsc.VectorSubcoreMesh(core_axis_name="c", subcore_axis_name="s"),
    scratch_types=[
        pltpu.VMEM((B_PER_W,), jnp.int32),
        pltpu.VMEM((B_PER_W, D), jnp.float32),
        pltpu.SemaphoreType.DMA,
    ],
)(_sc_gather_body)


LOG2E = math.log2(math.e)


def _oim_body(x_ref, feats_ref, se_ref, xs_scr, se_scr):
    i = pl.program_id(0)

    @pl.when(i == 0)
    def _init():
        x = x_ref[...]
        nrm = jnp.sqrt(jnp.sum(x * x, axis=1, keepdims=True))
        xs_scr[...] = (x * ((S * LOG2E) / jnp.clip(nrm, 1e-12))
                       ).astype(jnp.bfloat16)
        se_scr[...] = jnp.zeros_like(se_scr)

    def _tree_sum_128(v):
        parts = [v[:, k * 128:(k + 1) * 128] for k in range(v.shape[1] // 128)]
        while len(parts) > 1:
            nxt = [parts[k] + parts[k + 1] for k in range(0, len(parts) - 1, 2)]
            if len(parts) % 2:
                nxt.append(parts[-1])
            parts = nxt
        return parts[0]

    fb = feats_ref[...].astype(jnp.bfloat16)
    xs = xs_scr[...]
    acc = None
    for c in range(N_BLK // CHUNK):
        yc = lax.dot_general(xs, fb[c * CHUNK:(c + 1) * CHUNK, :],
                             (((1,), (1,)), ((), ())),
                             preferred_element_type=jnp.float32)
        ec = jnp.exp2(yc.astype(jnp.bfloat16))
        if (K_GRID - 1) * N_BLK + (c + 1) * CHUNK > C:
            col = (i * N_BLK + c * CHUNK
                   + lax.broadcasted_iota(jnp.int32, (B, CHUNK), 1))
            ec = jnp.where(col < C, ec, jnp.bfloat16(0.0))
        part = _tree_sum_128(ec)
        acc = part if acc is None else acc + part
    se_scr[...] += acc.astype(jnp.float32)

    @pl.when(i == K_GRID - 1)
    def _writeback():
        se_ref[...] = se_scr[...]


def _epilogue_body(x_ref, ft_ref, se_ref, out_ref):
    x = x_ref[...]
    xn = x / jnp.clip(jnp.sqrt(jnp.sum(x * x, axis=1, keepdims=True)),
                      1e-12)
    cos_t = jnp.sum(xn * ft_ref[...], axis=1, keepdims=True)
    sine = jnp.sqrt(jnp.clip(1.0 - cos_t * cos_t, 0.0, 1.0))
    phi = cos_t * COS_M - sine * SIN_M
    phi = jnp.where(cos_t > TH, phi, cos_t - MM)
    se = (jnp.sum(se_ref[...], axis=1, keepdims=True)
          + jnp.exp(S * phi) - jnp.exp(S * cos_t))
    lse = jnp.log(se)
    loss = jnp.mean(lse - S * phi)
    out_ref[...] = jnp.full((1, 1), loss, dtype=jnp.float32)


def kernel(inputs, targets, features):
    ft = _sc_gather(features, targets)
    se = pl.pallas_call(
        _oim_body,
        grid=(K_GRID,),
        in_specs=[
            pl.BlockSpec((B, D), lambda i: (0, 0)),
            pl.BlockSpec((N_BLK, D), lambda i: (i, 0)),
        ],
        out_specs=pl.BlockSpec((B, 128), lambda i: (0, 0)),
        out_shape=jax.ShapeDtypeStruct((B, 128), jnp.float32),
        scratch_shapes=[
            pltpu.VMEM((B, D), jnp.bfloat16),
            pltpu.VMEM((B, 128), jnp.float32),
        ],
        compiler_params=pltpu.CompilerParams(
            dimension_semantics=("arbitrary",),
        ),
    )(inputs, features)
    out = pl.pallas_call(
        _epilogue_body,
        out_shape=jax.ShapeDtypeStruct((1, 1), jnp.float32),
    )(inputs, ft, se)
    return out[0, 0]
